# Initial kernel scaffold; baseline (speedup 1.0000x reference)
import functools

import jax
import jax.numpy as jnp
from jax import lax
from jax.experimental import pallas as pl
from jax.experimental.pallas import tpu as pltpu
from jax.experimental.pallas import tpu_sc as plsc

VOCAB = 1000000
EMBED_DIM = 32
BATCH = 16384
HIST = 50

NUM_CORES = 2
NUM_SUBCORES = 16
NW = NUM_CORES * NUM_SUBCORES

B = BATCH * HIST
B_PER_W = B // NW
CHUNK = 3200
N_CHUNKS = B_PER_W // CHUNK


def _make_gather():
  mesh = plsc.VectorSubcoreMesh(
      core_axis_name="c", subcore_axis_name="s",
      num_cores=NUM_CORES, num_subcores=NUM_SUBCORES)

  @functools.partial(
      pl.kernel,
      out_type=jax.ShapeDtypeStruct((B, EMBED_DIM), jnp.float32),
      mesh=mesh,
      scratch_types=[
          pltpu.VMEM((CHUNK,), jnp.int32),
          pltpu.VMEM((CHUNK, EMBED_DIM), jnp.float32),
          pltpu.SemaphoreType.DMA,
      ],
  )
  def gather_kernel(idx_hbm, table_hbm, out_hbm, idx_v, rows_v, sem):
    wid = lax.axis_index("s") * NUM_CORES + lax.axis_index("c")
    wbase = wid * B_PER_W

    def body(i, carry):
      base = wbase + i * CHUNK
      pltpu.sync_copy(idx_hbm.at[pl.ds(base, CHUNK)], idx_v)
      pltpu.async_copy(table_hbm.at[idx_v], rows_v, sem).wait()
      pltpu.sync_copy(rows_v, out_hbm.at[pl.ds(base, CHUNK)])
      return carry

    lax.fori_loop(0, N_CHUNKS, body, 0)

  return gather_kernel


_gather = _make_gather()


@jax.jit
def kernel(x, table):
  flat_idx = x.reshape(B)
  out = _gather(flat_idx, table)
  return out.reshape(BATCH, HIST, EMBED_DIM)


# speedup vs baseline: 1.1113x; 1.1113x over previous
import functools

import jax
import jax.numpy as jnp
from jax import lax
from jax.experimental import pallas as pl
from jax.experimental.pallas import tpu as pltpu
from jax.experimental.pallas import tpu_sc as plsc

VOCAB = 1000000
EMBED_DIM = 32
BATCH = 16384
HIST = 50

NUM_CORES = 2
NUM_SUBCORES = 16
NW = NUM_CORES * NUM_SUBCORES

B = BATCH * HIST
B_PER_W = B // NW
CHUNK = 3200
N_CHUNKS = B_PER_W // CHUNK


def _make_gather():
  mesh = plsc.VectorSubcoreMesh(
      core_axis_name="c", subcore_axis_name="s",
      num_cores=NUM_CORES, num_subcores=NUM_SUBCORES)

  @functools.partial(
      pl.kernel,
      out_type=jax.ShapeDtypeStruct((B, EMBED_DIM), jnp.float32),
      mesh=mesh,
      scratch_types=[
          pltpu.VMEM((CHUNK,), jnp.int32),
          pltpu.VMEM((CHUNK, EMBED_DIM), jnp.float32),
          pltpu.SemaphoreType.DMA,
      ],
      compiler_params=pltpu.CompilerParams(use_tc_tiling_on_sc=False),
  )
  def gather_kernel(idx_hbm, table_hbm, out_hbm, idx_v, rows_v, sem):
    wid = lax.axis_index("s") * NUM_CORES + lax.axis_index("c")
    wbase = wid * B_PER_W

    def body(i, carry):
      base = wbase + i * CHUNK
      pltpu.sync_copy(idx_hbm.at[pl.ds(base, CHUNK)], idx_v)
      pltpu.async_copy(table_hbm.at[idx_v], rows_v, sem).wait()
      pltpu.sync_copy(rows_v, out_hbm.at[pl.ds(base, CHUNK)])
      return carry

    lax.fori_loop(0, N_CHUNKS, body, 0)

  return gather_kernel


_gather = _make_gather()


@jax.jit
def kernel(x, table):
  flat_idx = x.reshape(B)
  out = _gather(flat_idx, table)
  return out.reshape(BATCH, HIST, EMBED_DIM)


# speedup vs baseline: 1.1138x; 1.0023x over previous
import functools

import jax
import jax.numpy as jnp
from jax import lax
from jax.experimental import pallas as pl
from jax.experimental.pallas import tpu as pltpu
from jax.experimental.pallas import tpu_sc as plsc

VOCAB = 1000000
EMBED_DIM = 32
BATCH = 16384
HIST = 50

NUM_CORES = 2
NUM_SUBCORES = 16
NW = NUM_CORES * NUM_SUBCORES

B = BATCH * HIST
B_PER_W = B // NW
CHUNK = 640
NBUF = 4
N_CHUNKS = B_PER_W // CHUNK
N_GROUPS = N_CHUNKS // NBUF


def _make_gather():
  mesh = plsc.VectorSubcoreMesh(
      core_axis_name="c", subcore_axis_name="s",
      num_cores=NUM_CORES, num_subcores=NUM_SUBCORES)

  @functools.partial(
      pl.kernel,
      out_type=jax.ShapeDtypeStruct((B, EMBED_DIM), jnp.float32),
      mesh=mesh,
      scratch_types=[
          pltpu.VMEM((B_PER_W,), jnp.int32),
          pltpu.VMEM((NBUF, CHUNK, EMBED_DIM), jnp.float32),
          pltpu.SemaphoreType.DMA,
          [pltpu.SemaphoreType.DMA] * NBUF,
          [pltpu.SemaphoreType.DMA] * NBUF,
      ],
      compiler_params=pltpu.CompilerParams(use_tc_tiling_on_sc=False),
  )
  def gather_kernel(idx_hbm, table_hbm, out_hbm, idx_v, rows_v, isem, gsems,
                    ssems):
    wid = lax.axis_index("s") * NUM_CORES + lax.axis_index("c")
    wbase = wid * B_PER_W

    pltpu.async_copy(idx_hbm.at[pl.ds(wbase, B_PER_W)], idx_v, isem).wait()

    def issue_gather(chunk, b):
      return pltpu.async_copy(
          table_hbm.at[idx_v.at[pl.ds(chunk * CHUNK, CHUNK)]],
          rows_v.at[b], gsems[b])

    def wait_gather(b):
      pltpu.make_async_copy(
          table_hbm.at[idx_v.at[pl.ds(0, CHUNK)]], rows_v.at[b],
          gsems[b]).wait()

    def issue_store(chunk, b):
      pltpu.async_copy(
          rows_v.at[b], out_hbm.at[pl.ds(wbase + chunk * CHUNK, CHUNK)],
          ssems[b])

    def wait_store(b):
      pltpu.make_async_copy(
          rows_v.at[b], out_hbm.at[pl.ds(wbase, CHUNK)], ssems[b]).wait()

    def group(q, carry):
      for b in range(NBUF):
        i = q * NBUF + b
        @pl.when(q > 0)
        def _():
          wait_store(b)
        issue_gather(i, b)
        pb = (b - 1) % NBUF
        if b > 0:
          wait_gather(pb)
          issue_store(i - 1, pb)
        else:
          @pl.when(q > 0)
          def _():
            wait_gather(pb)
            issue_store(i - 1, pb)
      return carry

    lax.fori_loop(0, N_GROUPS, group, 0)

    last = NBUF - 1
    wait_gather(last)
    issue_store(N_CHUNKS - 1, last)
    for b in range(NBUF):
      wait_store(b)

  return gather_kernel


_gather = _make_gather()


@jax.jit
def kernel(x, table):
  flat_idx = x.reshape(B)
  out = _gather(flat_idx, table)
  return out.reshape(BATCH, HIST, EMBED_DIM)


# speedup vs baseline: 2.5429x; 2.2831x over previous
import functools

import jax
import jax.numpy as jnp
from jax import lax
from jax.experimental import pallas as pl
from jax.experimental.pallas import tpu as pltpu
from jax.experimental.pallas import tpu_sc as plsc

VOCAB = 1000000
EMBED_DIM = 32
BATCH = 16384
HIST = 50
HIST_PAD = 56
EMBED_PAD = 128

NUM_CORES = 2
NUM_SUBCORES = 16
NW = NUM_CORES * NUM_SUBCORES

B = BATCH * HIST
B_PER_W = B // NW
BATCH_PER_W = BATCH // NW
CHUNK_B = 8
CHUNK = CHUNK_B * HIST
NBUF = 4
N_CHUNKS = BATCH_PER_W // CHUNK_B
N_GROUPS = N_CHUNKS // NBUF


def _make_gather():
  mesh = plsc.VectorSubcoreMesh(
      core_axis_name="c", subcore_axis_name="s",
      num_cores=NUM_CORES, num_subcores=NUM_SUBCORES)

  @functools.partial(
      pl.kernel,
      out_type=jax.ShapeDtypeStruct((BATCH, HIST_PAD, EMBED_PAD),
                                    jnp.float32),
      mesh=mesh,
      scratch_types=[
          pltpu.VMEM((B_PER_W,), jnp.int32),
          pltpu.VMEM((NBUF, CHUNK, EMBED_DIM), jnp.float32),
          pltpu.SemaphoreType.DMA,
          [pltpu.SemaphoreType.DMA] * NBUF,
          [pltpu.SemaphoreType.DMA] * NBUF,
      ],
      compiler_params=pltpu.CompilerParams(use_tc_tiling_on_sc=False),
  )
  def gather_kernel(idx_hbm, table_hbm, out_hbm, idx_v, rows_v, isem, gsems,
                    ssems):
    wid = lax.axis_index("s") * NUM_CORES + lax.axis_index("c")
    wbase = wid * B_PER_W
    wbatch = wid * BATCH_PER_W

    pltpu.async_copy(idx_hbm.at[pl.ds(wbase, B_PER_W)], idx_v, isem).wait()

    def issue_gather(chunk, b):
      pltpu.async_copy(
          table_hbm.at[idx_v.at[pl.ds(chunk * CHUNK, CHUNK)]],
          rows_v.at[b], gsems[b])

    def wait_gather(b):
      pltpu.make_async_copy(
          table_hbm.at[idx_v.at[pl.ds(0, CHUNK)]], rows_v.at[b],
          gsems[b]).wait()

    def issue_store(chunk, b):
      for k in range(CHUNK_B):
        pltpu.async_copy(
            rows_v.at[b, pl.ds(k * HIST, HIST)],
            out_hbm.at[wbatch + chunk * CHUNK_B + k, pl.ds(0, HIST),
                       pl.ds(0, EMBED_DIM)],
            ssems[b])

    def wait_store(b):
      for k in range(CHUNK_B):
        pltpu.make_async_copy(
            rows_v.at[b, pl.ds(k * HIST, HIST)],
            out_hbm.at[wbatch, pl.ds(0, HIST), pl.ds(0, EMBED_DIM)],
            ssems[b]).wait()

    def group(q, carry):
      for b in range(NBUF):
        i = q * NBUF + b
        @pl.when(q > 0)
        def _():
          wait_store(b)
        issue_gather(i, b)
        pb = (b - 1) % NBUF
        if b > 0:
          wait_gather(pb)
          issue_store(i - 1, pb)
        else:
          @pl.when(q > 0)
          def _():
            wait_gather(pb)
            issue_store(i - 1, pb)
      return carry

    lax.fori_loop(0, N_GROUPS, group, 0)

    last = NBUF - 1
    wait_gather(last)
    issue_store(N_CHUNKS - 1, last)
    for b in range(NBUF):
      wait_store(b)

  return gather_kernel


_gather = _make_gather()


@jax.jit
def kernel(x, table):
  flat_idx = x.reshape(B)
  out = _gather(flat_idx, table)
  return out[:, :HIST, :EMBED_DIM]
